# Initial kernel scaffold; baseline (speedup 1.0000x reference)
import functools

import jax
import jax.numpy as jnp
from jax import lax
from jax.experimental import pallas as pl
from jax.experimental.pallas import tpu as pltpu
from jax.experimental.pallas import tpu_sc as plsc

NUM_RADIAL = 16
HIDDEN = 256
INT_EMB = 128
N_NODES = 10000
N_EDGES = 160000

HW = INT_EMB + 16
NC, NS = 2, 16
NW = NC * NS
CHUNK = 128
TOTAL_CHUNKS = N_EDGES // CHUNK
BASE_CHUNKS = TOTAL_CHUNKS // NW
EXTRA = TOTAL_CHUNKS - BASE_CHUNKS * NW
IDX_ROWS = BASE_CHUNKS + 1

BE = 2000
BN = 2000


def _h_body(ea_ref, w1t_ref, b1_ref, h_ref):
    a = jnp.dot(ea_ref[...], w1t_ref[...], preferred_element_type=jnp.float32)
    a = a + b1_ref[...]
    a = a * jax.nn.sigmoid(a)
    h_ref[...] = jnp.concatenate(
        [a, jnp.ones((a.shape[0], 16), jnp.float32)], axis=1)


def _out_body(acc_ref, x_ref, w2t_ref, b2_ref, o_ref):
    a = acc_ref[0] + acc_ref[1]
    nh = a[:, :INT_EMB]
    deg = a[:, INT_EMB:INT_EMB + 1]
    o_ref[...] = (x_ref[...]
                  + jnp.dot(nh, w2t_ref[...], preferred_element_type=jnp.float32)
                  + deg * b2_ref[...])


_vmesh = plsc.VectorSubcoreMesh(core_axis_name="c", subcore_axis_name="s")


@functools.partial(
    pl.kernel,
    out_type=jax.ShapeDtypeStruct((NC, N_NODES, HW), jnp.float32),
    mesh=_vmesh,
    scratch_types=[
        pltpu.VMEM((IDX_ROWS, CHUNK), jnp.int32),
        pltpu.VMEM((CHUNK, HW), jnp.float32),
        pltpu.VMEM_SHARED((N_NODES, HW), jnp.float32),
    ],
)
def _scatter_kernel(h_hbm, idx_hbm, zero_hbm, out_hbm, idx_v, h_v, acc_sh):
    c = lax.axis_index("c")
    s = lax.axis_index("s")
    wid = c * NS + s

    @pl.when(s < 10)
    def _():
        pltpu.sync_copy(zero_hbm.at[pl.ds(s * 1000, 1000)],
                        acc_sh.at[pl.ds(s * 1000, 1000)])

    plsc.subcore_barrier()

    start = wid * BASE_CHUNKS + jnp.minimum(wid, EXTRA)
    pltpu.sync_copy(idx_hbm.at[pl.ds(start, IDX_ROWS)], idx_v)

    @pl.loop(0, BASE_CHUNKS)
    def _(j):
        e0 = (start + j) * CHUNK
        pltpu.sync_copy(h_hbm.at[pl.ds(e0, CHUNK)], h_v)
        pltpu.sync_copy(h_v, acc_sh.at[idx_v.at[j]], add=True)

    @pl.when(wid < EXTRA)
    def _():
        e0 = (start + BASE_CHUNKS) * CHUNK
        pltpu.sync_copy(h_hbm.at[pl.ds(e0, CHUNK)], h_v)
        pltpu.sync_copy(h_v, acc_sh.at[idx_v.at[BASE_CHUNKS]], add=True)

    plsc.subcore_barrier()

    @pl.when(s < 10)
    def _():
        pltpu.sync_copy(acc_sh.at[pl.ds(s * 1000, 1000)],
                        out_hbm.at[c].at[pl.ds(s * 1000, 1000)])


def kernel(x, edge_index, edge_attr, W1, b1, W2, b2):
    col = edge_index[1].astype(jnp.int32)
    idx2d = col.reshape(TOTAL_CHUNKS, CHUNK)
    pad_rows = NW * IDX_ROWS - TOTAL_CHUNKS
    idx2d = jnp.concatenate(
        [idx2d, jnp.zeros((pad_rows, CHUNK), jnp.int32)], axis=0)

    w1t = W1.T
    b1r = b1.reshape(1, INT_EMB)
    w2t = W2.T
    b2r = b2.reshape(1, HIDDEN)

    h = pl.pallas_call(
        _h_body,
        grid=(N_EDGES // BE,),
        in_specs=[
            pl.BlockSpec((BE, NUM_RADIAL), lambda i: (i, 0)),
            pl.BlockSpec((NUM_RADIAL, INT_EMB), lambda i: (0, 0)),
            pl.BlockSpec((1, INT_EMB), lambda i: (0, 0)),
        ],
        out_specs=pl.BlockSpec((BE, HW), lambda i: (i, 0)),
        out_shape=jax.ShapeDtypeStruct((N_EDGES, HW), jnp.float32),
    )(edge_attr, w1t, b1r)

    zero = jnp.zeros((N_NODES, HW), jnp.float32)
    acc = _scatter_kernel(h, idx2d, zero)

    out = pl.pallas_call(
        _out_body,
        grid=(N_NODES // BN,),
        in_specs=[
            pl.BlockSpec((NC, BN, HW), lambda i: (0, i, 0)),
            pl.BlockSpec((BN, HIDDEN), lambda i: (i, 0)),
            pl.BlockSpec((INT_EMB, HIDDEN), lambda i: (0, 0)),
            pl.BlockSpec((1, HIDDEN), lambda i: (0, 0)),
        ],
        out_specs=pl.BlockSpec((BN, HIDDEN), lambda i: (i, 0)),
        out_shape=jax.ShapeDtypeStruct((N_NODES, HIDDEN), jnp.float32),
    )(acc, x, w2t, b2r)
    return out


# speedup vs baseline: 3.1698x; 3.1698x over previous
import dataclasses
import functools

import jax
import jax.numpy as jnp
import numpy as np
from jax import lax
from jax.experimental import pallas as pl
from jax.experimental.pallas import tpu as pltpu
from jax.experimental.pallas import tpu_sc as plsc

NUM_RADIAL = 16
HIDDEN = 256
INT_EMB = 128
N_NODES = 10000
N_EDGES = 160000

HW = INT_EMB
NC, NS = 2, 16
NW = NC * NS
CHUNK = 128
TOTAL_CHUNKS = N_EDGES // CHUNK
BASE_CHUNKS = TOTAL_CHUNKS // NW
EXTRA = TOTAL_CHUNKS - BASE_CHUNKS * NW
IDX_ROWS = 40

_CHUNK_LAYOUT = np.zeros(NW * IDX_ROWS, np.int32)
for _w in range(NW):
    _st = _w * BASE_CHUNKS + min(_w, EXTRA)
    _cnt = BASE_CHUNKS + (1 if _w < EXTRA else 0)
    for _j in range(IDX_ROWS):
        _CHUNK_LAYOUT[_w * IDX_ROWS + _j] = _st + _j if _j < _cnt else 0

BE = 2000
BN = 2000


def _h_body(ea_ref, w1t_ref, b1_ref, h_ref):
    a = jnp.dot(ea_ref[...], w1t_ref[...], preferred_element_type=jnp.float32)
    a = a + b1_ref[...]
    h_ref[...] = a * jax.nn.sigmoid(a)


def _out_body(acc_ref, deg_ref, x_ref, w2t_ref, b2_ref, o_ref):
    nh = acc_ref[0] + acc_ref[1]
    deg = jnp.sum(deg_ref[...], axis=1, keepdims=True)
    o_ref[...] = (x_ref[...]
                  + jnp.dot(nh, w2t_ref[...], preferred_element_type=jnp.float32)
                  + deg * b2_ref[...])


_vmesh = plsc.VectorSubcoreMesh(core_axis_name="c", subcore_axis_name="s")

_sc_params = pltpu.CompilerParams()
if "needs_layout_passes" in pltpu.CompilerParams.__dataclass_fields__:
    _sc_params = dataclasses.replace(_sc_params, needs_layout_passes=False)


@functools.partial(
    pl.kernel,
    out_type=(
        jax.ShapeDtypeStruct((NC, N_NODES, HW), jnp.float32),
        jax.ShapeDtypeStruct((NW * N_NODES,), jnp.float32),
    ),
    mesh=_vmesh,
    compiler_params=_sc_params,
    scratch_types=[
        pltpu.VMEM((IDX_ROWS, CHUNK), jnp.int32),
        pltpu.VMEM((CHUNK, HW), jnp.float32),
        pltpu.VMEM((N_NODES,), jnp.float32),
        pltpu.VMEM_SHARED((N_NODES, HW), jnp.float32),
    ],
)
def _scatter_kernel(h_hbm, idx_hbm, zero_hbm, out_hbm, deg_hbm,
                    idx_v, h_v, deg_v, acc_sh):
    c = lax.axis_index("c")
    s = lax.axis_index("s")
    wid = c * NS + s

    @pl.when(s < 10)
    def _():
        pltpu.sync_copy(zero_hbm.at[pl.ds(s * 1000, 1000)],
                        acc_sh.at[pl.ds(s * 1000, 1000)])

    zeros16 = jnp.zeros((16,), jnp.float32)
    @pl.loop(0, N_NODES // 16)
    def _(i):
        deg_v[pl.ds(i * 16, 16)] = zeros16

    plsc.subcore_barrier()

    start = wid * BASE_CHUNKS + jnp.minimum(wid, EXTRA)
    pltpu.sync_copy(idx_hbm.at[pl.ds(wid * IDX_ROWS, IDX_ROWS)], idx_v)

    ones16 = jnp.ones((16,), jnp.float32)

    def do_chunk(j):
        e0 = (start + j) * CHUNK
        pltpu.sync_copy(h_hbm.at[pl.ds(e0, CHUNK)], h_v)
        pltpu.sync_copy(h_v, acc_sh.at[idx_v.at[j]], add=True)
        @pl.loop(0, CHUNK // 16)
        def _(k):
            idx16 = idx_v[j, pl.ds(k * 16, 16)]
            plsc.addupdate_scatter(deg_v, [idx16], ones16)

    @pl.loop(0, BASE_CHUNKS)
    def _(j):
        do_chunk(j)

    @pl.when(wid < EXTRA)
    def _():
        do_chunk(BASE_CHUNKS)

    pltpu.sync_copy(deg_v, deg_hbm.at[pl.ds(wid * N_NODES, N_NODES)])

    plsc.subcore_barrier()

    @pl.when(s < 10)
    def _():
        pltpu.sync_copy(acc_sh.at[pl.ds(s * 1000, 1000)],
                        out_hbm.at[c].at[pl.ds(s * 1000, 1000)])


def kernel(x, edge_index, edge_attr, W1, b1, W2, b2):
    col = edge_index[1].astype(jnp.int32)
    idx2d = col.reshape(TOTAL_CHUNKS, CHUNK)[_CHUNK_LAYOUT]

    w1t = W1.T
    b1r = b1.reshape(1, INT_EMB)
    w2t = W2.T
    b2r = b2.reshape(1, HIDDEN)

    h = pl.pallas_call(
        _h_body,
        grid=(N_EDGES // BE,),
        in_specs=[
            pl.BlockSpec((BE, NUM_RADIAL), lambda i: (i, 0)),
            pl.BlockSpec((NUM_RADIAL, INT_EMB), lambda i: (0, 0)),
            pl.BlockSpec((1, INT_EMB), lambda i: (0, 0)),
        ],
        out_specs=pl.BlockSpec((BE, HW), lambda i: (i, 0)),
        out_shape=jax.ShapeDtypeStruct((N_EDGES, HW), jnp.float32),
    )(edge_attr, w1t, b1r)

    zero = jnp.zeros((N_NODES, HW), jnp.float32)
    acc, deg = _scatter_kernel(h, idx2d, zero)
    deg2d = deg.reshape(NW, N_NODES).T

    out = pl.pallas_call(
        _out_body,
        grid=(N_NODES // BN,),
        in_specs=[
            pl.BlockSpec((NC, BN, HW), lambda i: (0, i, 0)),
            pl.BlockSpec((BN, NW), lambda i: (i, 0)),
            pl.BlockSpec((BN, HIDDEN), lambda i: (i, 0)),
            pl.BlockSpec((INT_EMB, HIDDEN), lambda i: (0, 0)),
            pl.BlockSpec((1, HIDDEN), lambda i: (0, 0)),
        ],
        out_specs=pl.BlockSpec((BN, HIDDEN), lambda i: (i, 0)),
        out_shape=jax.ShapeDtypeStruct((N_NODES, HIDDEN), jnp.float32),
    )(acc, deg2d, x, w2t, b2r)
    return out


# speedup vs baseline: 3.3521x; 1.0575x over previous
import dataclasses
import functools

import jax
import jax.numpy as jnp
from jax import lax
from jax.experimental import pallas as pl
from jax.experimental.pallas import tpu as pltpu
from jax.experimental.pallas import tpu_sc as plsc

NUM_RADIAL = 16
HIDDEN = 256
INT_EMB = 128
N_NODES = 10000
N_EDGES = 160000

HW = INT_EMB
NC, NS = 2, 16
NW = NC * NS
CHUNK = 128
CPW = 40
PAIRS = CPW // 2
E_PAD = NW * CPW * CHUNK
N_PAD = 10240

BE = 2048
BN = 2000


def _h_body(ea_ref, w1t_ref, b1_ref, h_ref):
    a = jnp.dot(ea_ref[...], w1t_ref[...], preferred_element_type=jnp.float32)
    a = a + b1_ref[...]
    h_ref[...] = a * jax.nn.sigmoid(a)


def _out_body(acc_ref, deg_ref, x_ref, w2t_ref, b2_ref, o_ref):
    nh = acc_ref[0] + acc_ref[1]
    deg = jnp.sum(deg_ref[...], axis=1, keepdims=True)
    o_ref[...] = (x_ref[...]
                  + jnp.dot(nh, w2t_ref[...], preferred_element_type=jnp.float32)
                  + deg * b2_ref[...])


_vmesh = plsc.VectorSubcoreMesh(core_axis_name="c", subcore_axis_name="s")

_sc_params = pltpu.CompilerParams()
if "needs_layout_passes" in pltpu.CompilerParams.__dataclass_fields__:
    _sc_params = dataclasses.replace(_sc_params, needs_layout_passes=False)


@functools.partial(
    pl.kernel,
    out_type=(
        jax.ShapeDtypeStruct((NC, N_NODES, HW), jnp.float32),
        jax.ShapeDtypeStruct((NW * N_NODES,), jnp.float32),
    ),
    mesh=_vmesh,
    compiler_params=_sc_params,
    scratch_types=[
        pltpu.VMEM((CPW, CHUNK), jnp.int32),
        pltpu.VMEM((CHUNK, HW), jnp.float32),
        pltpu.VMEM((CHUNK, HW), jnp.float32),
        pltpu.VMEM((N_PAD,), jnp.float32),
        pltpu.VMEM_SHARED((N_PAD, HW), jnp.float32),
        pltpu.SemaphoreType.DMA,
        pltpu.SemaphoreType.DMA,
    ],
)
def _scatter_kernel(h_hbm, idx_hbm, zero_hbm, out_hbm, deg_hbm,
                    idx_v, h_a, h_b, deg_v, acc_sh, sem_a, sem_b):
    c = lax.axis_index("c")
    s = lax.axis_index("s")
    wid = c * NS + s

    pltpu.sync_copy(zero_hbm.at[pl.ds(s * 640, 640)],
                    acc_sh.at[pl.ds(s * 640, 640)])

    zeros16 = jnp.zeros((16,), jnp.float32)
    @pl.loop(0, N_PAD // 16)
    def _(i):
        deg_v[pl.ds(i * 16, 16)] = zeros16

    plsc.subcore_barrier()

    base_chunk = wid * CPW
    e_base = base_chunk * CHUNK
    pltpu.sync_copy(idx_hbm.at[pl.ds(base_chunk, CPW)], idx_v)

    ones16 = jnp.ones((16,), jnp.float32)

    def deg_update(j):
        @pl.loop(0, CHUNK // 16)
        def _(k):
            idx16 = idx_v[j, pl.ds(k * 16, 16)]
            plsc.addupdate_scatter(deg_v, [idx16], ones16)

    def load(j, buf, sem):
        pltpu.make_async_copy(
            h_hbm.at[pl.ds(e_base + j * CHUNK, CHUNK)], buf, sem).start()

    def drain_load(buf, sem):
        pltpu.make_async_copy(h_hbm.at[pl.ds(0, CHUNK)], buf, sem).wait()

    load(0, h_a, sem_a)

    @pl.loop(0, PAIRS)
    def _(t):
        c0 = 2 * t
        c1 = c0 + 1
        load(c1, h_b, sem_b)
        drain_load(h_a, sem_a)
        sc_a = pltpu.async_copy(h_a, acc_sh.at[idx_v.at[c0]], sem_a, add=True)
        deg_update(c0)
        drain_load(h_b, sem_b)
        sc_a.wait()

        @pl.when(t < PAIRS - 1)
        def _():
            load(c0 + 2, h_a, sem_a)

        sc_b = pltpu.async_copy(h_b, acc_sh.at[idx_v.at[c1]], sem_b, add=True)
        deg_update(c1)
        sc_b.wait()

    pltpu.sync_copy(deg_v.at[pl.ds(0, N_NODES)],
                    deg_hbm.at[pl.ds(wid * N_NODES, N_NODES)])

    plsc.subcore_barrier()

    @pl.when(s < 10)
    def _():
        pltpu.sync_copy(acc_sh.at[pl.ds(s * 1000, 1000)],
                        out_hbm.at[c].at[pl.ds(s * 1000, 1000)])


def kernel(x, edge_index, edge_attr, W1, b1, W2, b2):
    col = edge_index[1].astype(jnp.int32)
    col_pad = jnp.full((E_PAD,), N_NODES, jnp.int32).at[:N_EDGES].set(col)
    idx2d = col_pad.reshape(NW * CPW, CHUNK)
    ea_pad = jnp.zeros((E_PAD, NUM_RADIAL), jnp.float32).at[:N_EDGES].set(edge_attr)

    w1t = W1.T
    b1r = b1.reshape(1, INT_EMB)
    w2t = W2.T
    b2r = b2.reshape(1, HIDDEN)

    h = pl.pallas_call(
        _h_body,
        grid=(E_PAD // BE,),
        in_specs=[
            pl.BlockSpec((BE, NUM_RADIAL), lambda i: (i, 0)),
            pl.BlockSpec((NUM_RADIAL, INT_EMB), lambda i: (0, 0)),
            pl.BlockSpec((1, INT_EMB), lambda i: (0, 0)),
        ],
        out_specs=pl.BlockSpec((BE, HW), lambda i: (i, 0)),
        out_shape=jax.ShapeDtypeStruct((E_PAD, HW), jnp.float32),
    )(ea_pad, w1t, b1r)

    zero = jnp.zeros((N_PAD, HW), jnp.float32)
    acc, deg = _scatter_kernel(h, idx2d, zero)
    deg2d = deg.reshape(NW, N_NODES).T

    out = pl.pallas_call(
        _out_body,
        grid=(N_NODES // BN,),
        in_specs=[
            pl.BlockSpec((NC, BN, HW), lambda i: (0, i, 0)),
            pl.BlockSpec((BN, NW), lambda i: (i, 0)),
            pl.BlockSpec((BN, HIDDEN), lambda i: (i, 0)),
            pl.BlockSpec((INT_EMB, HIDDEN), lambda i: (0, 0)),
            pl.BlockSpec((1, HIDDEN), lambda i: (0, 0)),
        ],
        out_specs=pl.BlockSpec((BN, HIDDEN), lambda i: (i, 0)),
        out_shape=jax.ShapeDtypeStruct((N_NODES, HIDDEN), jnp.float32),
    )(acc, deg2d, x, w2t, b2r)
    return out
